# Initial kernel scaffold; baseline (speedup 1.0000x reference)
import jax
import jax.numpy as jnp
from jax.experimental import pallas as pl


def kernel(x, edge_index, W_msg, b_msg, weight_ih, weight_hh, bias_ih, bias_hh):
    raise NotImplementedError("write your pallas kernel here")


# speedup vs baseline: 3.7241x; 3.7241x over previous
import functools

import jax
import jax.numpy as jnp
from jax import lax
from jax.experimental import pallas as pl
from jax.experimental.pallas import tpu as pltpu
from jax.experimental.pallas import tpu_sc as plsc

H = 128
N = 10000
E = 320000
NUM_STEPS = 5

NC = 2
NS = 16
NW = NC * NS
CH = 128
EPW = 10112
NCHUNK = EPW // CH
EPAD = EPW * NW
NPAD = 10240
RPT = NPAD // NS
DW = 16

_sc_mesh = plsc.VectorSubcoreMesh(core_axis_name="c", subcore_axis_name="s")


def _seg_body(row_hbm, col_hbm, h_hbm, zeros_hbm, out_hbm,
              col_v, row_v, rows_v, acc_sh, gsem):
  c = lax.axis_index("c")
  s = lax.axis_index("s")
  wid = c * NS + s
  base = wid * EPW

  pltpu.sync_copy(zeros_hbm, rows_v)
  for k in range(RPT // CH):
    pltpu.sync_copy(rows_v, acc_sh.at[pl.ds(s * RPT + k * CH, CH)])
  plsc.subcore_barrier()

  def chunk(j, carry):
    off = pl.multiple_of(base + j * CH, 8)
    pltpu.sync_copy(col_hbm.at[pl.ds(off, CH)], col_v)
    pltpu.sync_copy(row_hbm.at[pl.ds(off, CH)], row_v)
    pltpu.async_copy(h_hbm.at[col_v], rows_v, gsem).wait()
    pltpu.sync_copy(rows_v, acc_sh.at[row_v], add=True)
    return carry

  lax.fori_loop(0, NCHUNK, chunk, 0)
  plsc.subcore_barrier()

  for k in range(RPT // CH):
    r0 = s * RPT + k * CH
    pltpu.sync_copy(acc_sh.at[pl.ds(r0, CH)], rows_v)
    pltpu.sync_copy(rows_v, out_hbm.at[c, pl.ds(r0, CH)])


_seg_call = functools.partial(
    pl.kernel,
    out_type=jax.ShapeDtypeStruct((NC, NPAD, H), jnp.float32),
    mesh=_sc_mesh,
    scratch_types=[
        pltpu.VMEM((CH,), jnp.int32),
        pltpu.VMEM((CH,), jnp.int32),
        pltpu.VMEM((CH, H), jnp.float32),
        pltpu.VMEM_SHARED((NPAD, H), jnp.float32),
        pltpu.SemaphoreType.DMA,
    ],
)(_seg_body)


def _deg_body(row_hbm, const_hbm, out_hbm, row_v, ones_v, deg_sh, gsem):
  c = lax.axis_index("c")
  s = lax.axis_index("s")
  wid = c * NS + s
  base = wid * EPW

  pltpu.sync_copy(const_hbm.at[pl.ds(0, CH)], ones_v)
  for k in range(RPT // CH):
    pltpu.sync_copy(ones_v, deg_sh.at[pl.ds(s * RPT + k * CH, CH)])
  pltpu.sync_copy(const_hbm.at[pl.ds(CH, CH)], ones_v)
  plsc.subcore_barrier()

  def chunk(j, carry):
    off = pl.multiple_of(base + j * CH, 8)
    pltpu.sync_copy(row_hbm.at[pl.ds(off, CH)], row_v)
    pltpu.sync_copy(ones_v, deg_sh.at[row_v], add=True)
    return carry

  lax.fori_loop(0, NCHUNK, chunk, 0)
  plsc.subcore_barrier()

  for k in range(RPT // CH):
    r0 = s * RPT + k * CH
    pltpu.sync_copy(deg_sh.at[pl.ds(r0, CH)], ones_v)
    pltpu.sync_copy(ones_v, out_hbm.at[c, pl.ds(r0, CH)])


_deg_call = functools.partial(
    pl.kernel,
    out_type=jax.ShapeDtypeStruct((NC, NPAD, DW), jnp.float32),
    mesh=_sc_mesh,
    scratch_types=[
        pltpu.VMEM((CH,), jnp.int32),
        pltpu.VMEM((CH, DW), jnp.float32),
        pltpu.VMEM_SHARED((NPAD, DW), jnp.float32),
        pltpu.SemaphoreType.DMA,
    ],
)(_deg_body)


BN = 1000


def _gru_block(sp_ref, dp_ref, h_ref, wm_ref, wih_ref, whh_ref,
               bm_ref, bih_ref, bhh_ref, out_ref):
  S = sp_ref[0] + sp_ref[1]
  deg = dp_ref[0, :, 0:1] + dp_ref[1, :, 0:1]
  h = h_ref[...]
  agg = jnp.dot(S, wm_ref[...], preferred_element_type=jnp.float32)
  agg = agg + deg * bm_ref[...]
  gi = jnp.dot(agg, wih_ref[...], preferred_element_type=jnp.float32)
  gi = gi + bih_ref[...]
  gh = jnp.dot(h, whh_ref[...], preferred_element_type=jnp.float32)
  gh = gh + bhh_ref[...]
  r = jax.nn.sigmoid(gi[:, :H] + gh[:, :H])
  z = jax.nn.sigmoid(gi[:, H:2 * H] + gh[:, H:2 * H])
  n = jnp.tanh(gi[:, 2 * H:] + r * gh[:, 2 * H:])
  out_ref[...] = (1.0 - z) * n + z * h


def _gru_call(Sp, degp, h, Wm_t, Wih_t, Whh_t, bm, bih, bhh):
  nb = N // BN
  full = lambda shape: pl.BlockSpec(shape, lambda i: (0,) * len(shape))
  return pl.pallas_call(
      _gru_block,
      grid=(nb,),
      in_specs=[
          pl.BlockSpec((NC, BN, H), lambda i: (0, i, 0)),
          pl.BlockSpec((NC, BN, DW), lambda i: (0, i, 0)),
          pl.BlockSpec((BN, H), lambda i: (i, 0)),
          full((H, H)),
          full((H, 3 * H)),
          full((H, 3 * H)),
          full((1, H)),
          full((1, 3 * H)),
          full((1, 3 * H)),
      ],
      out_specs=pl.BlockSpec((BN, H), lambda i: (i, 0)),
      out_shape=jax.ShapeDtypeStruct((N, H), jnp.float32),
  )(Sp, degp, h, Wm_t, Wih_t, Whh_t, bm, bih, bhh)


def kernel(x, edge_index, W_msg, b_msg, weight_ih, weight_hh, bias_ih, bias_hh):
  row = edge_index[0].astype(jnp.int32)
  col = edge_index[1].astype(jnp.int32)
  pad = EPAD - E
  rowp = jnp.concatenate([row, jnp.full((pad,), N, jnp.int32)])
  colp = jnp.concatenate([col, jnp.zeros((pad,), jnp.int32)])
  zeros_hbm = jnp.zeros((CH, H), jnp.float32)
  const16 = jnp.concatenate(
      [jnp.zeros((CH, DW), jnp.float32), jnp.ones((CH, DW), jnp.float32)])

  degp = _deg_call(rowp, const16)

  Wm_t = W_msg.T
  Wih_t = weight_ih.T
  Whh_t = weight_hh.T
  bm = b_msg.reshape(1, H)
  bih = bias_ih.reshape(1, 3 * H)
  bhh = bias_hh.reshape(1, 3 * H)

  h = x
  for _ in range(NUM_STEPS):
    Sp = _seg_call(rowp, colp, h, zeros_hbm)
    h = _gru_call(Sp, degp, h, Wm_t, Wih_t, Whh_t, bm, bih, bhh)
  return h
